# baseline (speedup 1.0000x reference)
import jax
import jax.numpy as jnp
from jax import lax
from jax.experimental import pallas as pl
from jax.experimental.pallas import tpu as pltpu
from jax.experimental.pallas import tpu_sc as plsc

N = 320000
F = 128
G = 256

_INFO = plsc.get_sparse_core_info()
NC = _INFO.num_cores
NS = _INFO.num_subcores
NW = NC * NS
CHUNK = 128
TOTCHUNK = N // CHUNK
NCHUNK = TOTCHUNK // NW
EXTRA = TOTCHUNK - NCHUNK * NW
ROWS_PER_W = NCHUNK * CHUNK
ROWS_PER_TILE_OUT = G // NS


def _sc_partial_kernel(x_hbm, b_hbm, out_hbm, xbufa, xbufb, ibuf, obuf,
                       acc0, acc1, sga, sgb, ssa, ssb):
    c = lax.axis_index("c")
    s = lax.axis_index("s")
    wid = s * NC + c
    base = wid * ROWS_PER_W

    for r in range(ROWS_PER_TILE_OUT):
        for j in range(F // 16):
            obuf[r, pl.ds(j * 16, 16)] = jnp.zeros((16,), jnp.float32)
    row0 = s * ROWS_PER_TILE_OUT
    pltpu.sync_copy(obuf, acc0.at[pl.ds(row0, ROWS_PER_TILE_OUT)])
    pltpu.sync_copy(obuf, acc1.at[pl.ds(row0, ROWS_PER_TILE_OUT)])
    pltpu.sync_copy(b_hbm.at[wid], ibuf)
    plsc.subcore_barrier()

    def gather_start(chunk_idx, buf, sem):
        pltpu.async_copy(
            x_hbm.at[pl.ds(base + chunk_idx * CHUNK, CHUNK)], buf, sem)

    def gather_wait(chunk_idx, buf, sem):
        pltpu.make_async_copy(
            x_hbm.at[pl.ds(base + chunk_idx * CHUNK, CHUNK)], buf, sem).wait()

    def scatter_start(chunk_idx, buf, acc, sem):
        pltpu.async_copy(buf, acc.at[ibuf.at[chunk_idx]], sem, add=True)

    def scatter_wait(chunk_idx, buf, acc, sem):
        pltpu.make_async_copy(buf, acc.at[ibuf.at[chunk_idx]], sem).wait()

    gather_start(0, xbufa, sga)
    gather_start(1, xbufb, sgb)

    def body(i, carry):
        c0 = 2 * i
        gather_wait(c0, xbufa, sga)
        scatter_start(c0, xbufa, acc0, ssa)
        gather_wait(c0 + 1, xbufb, sgb)
        scatter_start(c0 + 1, xbufb, acc1, ssb)
        nxt = jnp.minimum(c0 + 2, NCHUNK)
        scatter_wait(c0, xbufa, acc0, ssa)
        gather_start(nxt, xbufa, sga)
        nxt2 = jnp.minimum(c0 + 3, NCHUNK)
        scatter_wait(c0 + 1, xbufb, acc1, ssb)
        gather_start(nxt2, xbufb, sgb)
        return carry

    lax.fori_loop(0, NCHUNK // 2, body, 0)
    gather_wait(NCHUNK, xbufa, sga)
    gather_wait(NCHUNK, xbufb, sgb)

    @pl.when(wid < EXTRA)
    def _():
        start = NW * ROWS_PER_W + wid * CHUNK
        pltpu.sync_copy(x_hbm.at[pl.ds(start, CHUNK)], xbufa)
        pltpu.sync_copy(xbufa, acc0.at[ibuf.at[NCHUNK]], add=True)

    plsc.subcore_barrier()

    pltpu.sync_copy(acc0.at[pl.ds(row0, ROWS_PER_TILE_OUT)], obuf)
    pltpu.sync_copy(obuf, out_hbm.at[c, 0, pl.ds(row0, ROWS_PER_TILE_OUT)])
    pltpu.sync_copy(acc1.at[pl.ds(row0, ROWS_PER_TILE_OUT)], obuf)
    pltpu.sync_copy(obuf, out_hbm.at[c, 1, pl.ds(row0, ROWS_PER_TILE_OUT)])


@jax.jit
def _sc_partials(x, batch_blocked):
    mesh = plsc.VectorSubcoreMesh(core_axis_name="c", subcore_axis_name="s")
    return pl.kernel(
        _sc_partial_kernel,
        mesh=mesh,
        out_type=jax.ShapeDtypeStruct((NC, 2, G, F), jnp.float32),
        scratch_types=[
            pltpu.VMEM((CHUNK, F), jnp.float32),
            pltpu.VMEM((CHUNK, F), jnp.float32),
            pltpu.VMEM((NCHUNK + 1, CHUNK), jnp.int32),
            pltpu.VMEM((ROWS_PER_TILE_OUT, F), jnp.float32),
            pltpu.VMEM_SHARED((G, F), jnp.float32),
            pltpu.VMEM_SHARED((G, F), jnp.float32),
            pltpu.SemaphoreType.DMA,
            pltpu.SemaphoreType.DMA,
            pltpu.SemaphoreType.DMA,
            pltpu.SemaphoreType.DMA,
        ],
    )(x, batch_blocked)


def _combine_kernel(p_ref, o_ref):
    o_ref[...] = (p_ref[0, 0] + p_ref[0, 1]) + (p_ref[1, 0] + p_ref[1, 1])


@jax.jit
def _combine(partials):
    return pl.pallas_call(
        _combine_kernel,
        out_shape=jax.ShapeDtypeStruct((G, F), jnp.float32),
    )(partials)


def kernel(x, batch):
    b = batch.astype(jnp.int32).reshape(TOTCHUNK, CHUNK)
    slabs = b[:NW * NCHUNK].reshape(NW, NCHUNK, CHUNK)
    extras = jnp.concatenate(
        [b[NW * NCHUNK:], jnp.zeros((NW - EXTRA, CHUNK), jnp.int32)]
    ).reshape(NW, 1, CHUNK)
    batch_blocked = jnp.concatenate([slabs, extras], axis=1)
    partials = _sc_partials(x, batch_blocked)
    return _combine(partials)


# speedup vs baseline: 1.1363x; 1.1363x over previous
import jax
import jax.numpy as jnp
from jax import lax
from jax.experimental import pallas as pl
from jax.experimental.pallas import tpu as pltpu
from jax.experimental.pallas import tpu_sc as plsc

N = 320000
F = 128
G = 256

_INFO = plsc.get_sparse_core_info()
NC = _INFO.num_cores
NS = _INFO.num_subcores
NW = NC * NS
CHUNK = 128
TOTCHUNK = N // CHUNK
NCHUNK = TOTCHUNK // NW
EXTRA = TOTCHUNK - NCHUNK * NW
ROWS_PER_W = NCHUNK * CHUNK
ROWS_PER_TILE_OUT = G // NS
NBUF = 4
DEPTH = 3


def _sc_partial_kernel(x_hbm, b_hbm, out_hbm, xb0, xb1, xb2, xb3, ibuf, obuf,
                       acc, sg0, sg1, sg2, sg3):
    c = lax.axis_index("c")
    s = lax.axis_index("s")
    wid = s * NC + c
    base = wid * ROWS_PER_W
    xbufs = (xb0, xb1, xb2, xb3)
    gsems = (sg0, sg1, sg2, sg3)

    for r in range(ROWS_PER_TILE_OUT):
        for j in range(F // 16):
            obuf[r, pl.ds(j * 16, 16)] = jnp.zeros((16,), jnp.float32)
    row0 = s * ROWS_PER_TILE_OUT
    pltpu.sync_copy(obuf, acc.at[pl.ds(row0, ROWS_PER_TILE_OUT)])
    pltpu.sync_copy(b_hbm.at[wid], ibuf)
    plsc.subcore_barrier()

    def gather_start(chunk_idx, b):
        pltpu.async_copy(
            x_hbm.at[pl.ds(base + chunk_idx * CHUNK, CHUNK)], xbufs[b], gsems[b])

    def gather_wait(chunk_idx, b):
        pltpu.make_async_copy(
            x_hbm.at[pl.ds(base + chunk_idx * CHUNK, CHUNK)], xbufs[b],
            gsems[b]).wait()

    def scatter_add(chunk_idx, b):
        pltpu.sync_copy(xbufs[b], acc.at[ibuf.at[chunk_idx]], add=True)

    for cc in range(DEPTH):
        gather_start(cc, cc)
    for cc in range(NBUF):
        gather_wait(cc, cc)
        scatter_add(cc, cc)
        gather_start(cc + DEPTH, (cc + DEPTH) % NBUF)

    def body(i, carry):
        c0 = NBUF * i
        for b in range(NBUF):
            cc = c0 + b
            gather_wait(cc, b)
            scatter_add(cc, b)

            @pl.when(cc + DEPTH < NCHUNK)
            def _():
                gather_start(cc + DEPTH, (b + DEPTH) % NBUF)
        return carry

    lax.fori_loop(1, (NCHUNK - 2) // NBUF, body, 0)
    gather_wait(NCHUNK - 2, (NCHUNK - 2) % NBUF)
    scatter_add(NCHUNK - 2, (NCHUNK - 2) % NBUF)
    gather_wait(NCHUNK - 1, (NCHUNK - 1) % NBUF)
    scatter_add(NCHUNK - 1, (NCHUNK - 1) % NBUF)

    @pl.when(wid < EXTRA)
    def _():
        start = NW * ROWS_PER_W + wid * CHUNK
        pltpu.sync_copy(x_hbm.at[pl.ds(start, CHUNK)], xbufs[0])
        pltpu.sync_copy(xbufs[0], acc.at[ibuf.at[NCHUNK]], add=True)

    plsc.subcore_barrier()

    pltpu.sync_copy(acc.at[pl.ds(row0, ROWS_PER_TILE_OUT)], obuf)
    pltpu.sync_copy(obuf, out_hbm.at[c, pl.ds(row0, ROWS_PER_TILE_OUT)])


@jax.jit
def _sc_partials(x, batch_blocked):
    mesh = plsc.VectorSubcoreMesh(core_axis_name="c", subcore_axis_name="s")
    return pl.kernel(
        _sc_partial_kernel,
        mesh=mesh,
        out_type=jax.ShapeDtypeStruct((NC, G, F), jnp.float32),
        scratch_types=[
            pltpu.VMEM((CHUNK, F), jnp.float32),
            pltpu.VMEM((CHUNK, F), jnp.float32),
            pltpu.VMEM((CHUNK, F), jnp.float32),
            pltpu.VMEM((CHUNK, F), jnp.float32),
            pltpu.VMEM((NCHUNK + 1, CHUNK), jnp.int32),
            pltpu.VMEM((ROWS_PER_TILE_OUT, F), jnp.float32),
            pltpu.VMEM_SHARED((G, F), jnp.float32),
            pltpu.SemaphoreType.DMA,
            pltpu.SemaphoreType.DMA,
            pltpu.SemaphoreType.DMA,
            pltpu.SemaphoreType.DMA,
        ],
    )(x, batch_blocked)


def _combine_kernel(p_ref, o_ref):
    o_ref[...] = p_ref[0] + p_ref[1]


@jax.jit
def _combine(partials):
    return pl.pallas_call(
        _combine_kernel,
        out_shape=jax.ShapeDtypeStruct((G, F), jnp.float32),
    )(partials)


def kernel(x, batch):
    b = batch.astype(jnp.int32).reshape(TOTCHUNK, CHUNK)
    slabs = b[:NW * NCHUNK].reshape(NW, NCHUNK, CHUNK)
    extras = jnp.concatenate(
        [b[NW * NCHUNK:], jnp.zeros((NW - EXTRA, CHUNK), jnp.int32)]
    ).reshape(NW, 1, CHUNK)
    batch_blocked = jnp.concatenate([slabs, extras], axis=1)
    partials = _sc_partials(x, batch_blocked)
    return _combine(partials)
